# baseline (speedup 1.0000x reference)
import functools

import jax
import jax.numpy as jnp
from jax import lax
from jax.experimental import pallas as pl
from jax.experimental.pallas import tpu as pltpu
from jax.experimental.pallas import tpu_sc as plsc

B = 8
T = 144
V = 25
NV = T * V
NTOT = B * NV
DH = 128
NC = 60
EPS = 1e-5
E = 57600

GF = B * DH
NH = NUM_AUX_1 = 9
RB = 400
NB = NV // RB
AGG_RT = 1200
EDGE_TILE = 11520
SC_RB = 24
_UNROLL = 8
SC_NCH = NV // SC_RB
_INTERPRET = False


def _build_adj(edge_index):
    if _INTERPRET:
        src, dst = edge_index[0], edge_index[1]
        a = jnp.zeros((NV, NV), jnp.float32).at[dst, src].add(1.0)
        return a + jnp.eye(NV, dtype=jnp.float32)

    mesh = plsc.VectorSubcoreMesh(core_axis_name="c", subcore_axis_name="s")

    @functools.partial(
        pl.kernel,
        mesh=mesh,
        out_type=jax.ShapeDtypeStruct((NV, NV), jnp.float32),
        compiler_params=pltpu.CompilerParams(
            use_tc_tiling_on_sc=False, needs_layout_passes=False),
        scratch_types=[
            pltpu.VMEM((SC_RB, NV), jnp.float32),
            pltpu.VMEM((EDGE_TILE,), jnp.int32),
            pltpu.VMEM((EDGE_TILE,), jnp.int32),
        ],
    )
    def build(edge_hbm, zrow_hbm, out_hbm, ablk, srcb, dstb):
        wid = lax.axis_index("s") * 2 + lax.axis_index("c")
        ones16 = jnp.full((16,), 1.0, jnp.float32)
        lane = lax.iota(jnp.int32, 16)
        n_workers = 32
        for p in range((SC_NCH + 31) // 32):
            c = wid + n_workers * p

            @pl.when(c < SC_NCH)
            def _():
                lo = c * SC_RB
                pltpu.sync_copy(zrow_hbm, ablk)
                for t in range(E // EDGE_TILE):
                    pltpu.sync_copy(
                        edge_hbm.at[0, pl.ds(t * EDGE_TILE, EDGE_TILE)], srcb)
                    pltpu.sync_copy(
                        edge_hbm.at[1, pl.ds(t * EDGE_TILE, EDGE_TILE)], dstb)

                    def body(v, carry):
                        base = v * (16 * _UNROLL)
                        for u in range(_UNROLL):
                            dv = dstb[pl.ds(base + u * 16, 16)]
                            sv = srcb[pl.ds(base + u * 16, 16)]
                            m = (dv >= lo) & (dv < lo + SC_RB)

                            @pl.when(jnp.any(m))
                            def _():
                                plsc.addupdate_scatter(
                                    ablk, [dv - lo, sv], ones16, mask=m)
                        return carry

                    lax.fori_loop(0, EDGE_TILE // (16 * _UNROLL), body, 0)
                for q in range((SC_RB + 15) // 16):
                    rr = lane + q * 16
                    plsc.addupdate_scatter(
                        ablk, [rr, rr + lo], ones16, mask=rr < SC_RB)
                pltpu.sync_copy(ablk, out_hbm.at[pl.ds(lo, SC_RB)])

    zrow = jnp.zeros((SC_RB, NV), jnp.float32)
    return build(edge_index, zrow)


def _dinv_kernel(a_ref, o_ref, ab_ref):
    a = a_ref[...]
    deg = jnp.sum(a, axis=1, keepdims=True)
    o_ref[...] = lax.rsqrt(deg)
    ab_ref[...] = a.astype(jnp.bfloat16)


def _dinv_call(a1):
    return pl.pallas_call(
        _dinv_kernel,
        grid=(NB,),
        in_specs=[pl.BlockSpec((RB, NV), lambda j: (j, 0))],
        out_specs=[
            pl.BlockSpec((RB, 1), lambda j: (j, 0)),
            pl.BlockSpec((RB, NV), lambda j: (j, 0)),
        ],
        out_shape=[
            jax.ShapeDtypeStruct((NV, 1), jnp.float32),
            jax.ShapeDtypeStruct((NV, NV), jnp.bfloat16),
        ],
        interpret=_INTERPRET,
    )(a1)


def _xw_first_kernel(x_ref, w_ref, dinv_ref, o_ref):
    xb = x_ref[0]
    y = jnp.dot(xb, w_ref[...], preferred_element_type=jnp.float32)
    o_ref[...] = (y * dinv_ref[...]).astype(jnp.bfloat16)


def _xw_first(x, w, dinv):
    return pl.pallas_call(
        _xw_first_kernel,
        grid=(B,),
        in_specs=[
            pl.BlockSpec((1, NV, 3), lambda g: (g, 0, 0)),
            pl.BlockSpec((3, DH), lambda g: (0, 0)),
            pl.BlockSpec((NV, 1), lambda g: (0, 0)),
        ],
        out_specs=pl.BlockSpec((NV, DH), lambda g: (0, g)),
        out_shape=jax.ShapeDtypeStruct((NV, GF), jnp.bfloat16),
        interpret=_INTERPRET,
    )(x, w, dinv)


def _xw_kernel(z_ref, st_ref, gam_ref, bet_ref, w_ref, dinv_ref, o_ref):
    st = st_ref[...]
    mean = st[0:1] * (1.0 / NTOT)
    var = st[1:2] * (1.0 / NTOT) - mean * mean
    inv = lax.rsqrt(var + EPS)
    a = (z_ref[...].astype(jnp.float32) - mean) * (inv * gam_ref[...])
    a = jnp.maximum(a + bet_ref[...], 0.0)
    y = jnp.dot(a, w_ref[...], preferred_element_type=jnp.float32)
    o_ref[...] = (y * dinv_ref[...]).astype(jnp.bfloat16)


def _xw_call(z, st, gam, bet, w, dinv):
    return pl.pallas_call(
        _xw_kernel,
        grid=(B,),
        in_specs=[
            pl.BlockSpec((NV, DH), lambda g: (0, g)),
            pl.BlockSpec((2, DH), lambda g: (0, 0)),
            pl.BlockSpec((1, DH), lambda g: (0, 0)),
            pl.BlockSpec((1, DH), lambda g: (0, 0)),
            pl.BlockSpec((DH, DH), lambda g: (0, 0)),
            pl.BlockSpec((NV, 1), lambda g: (0, 0)),
        ],
        out_specs=pl.BlockSpec((NV, DH), lambda g: (0, g)),
        out_shape=jax.ShapeDtypeStruct((NV, GF), jnp.bfloat16),
        interpret=_INTERPRET,
    )(z, st, gam, bet, w, dinv)


def _xwh_kernel(z_ref, st_ref, gam_ref, bet_ref, w_ref, dinv_ref, o_ref):
    st = st_ref[...]
    mean = st[0:1] * (1.0 / NTOT)
    var = st[1:2] * (1.0 / NTOT) - mean * mean
    inv = lax.rsqrt(var + EPS)
    a = (z_ref[...].astype(jnp.float32) - mean) * (inv * gam_ref[...])
    a = jnp.maximum(a + bet_ref[...], 0.0)
    y = jnp.dot(a, w_ref[0], preferred_element_type=jnp.float32)
    y = pltpu.roll(y, lax.rem(NV - RB * pl.program_id(0), NV), axis=0)
    o_ref[0] = (y * dinv_ref[...]).astype(jnp.bfloat16)


def _xwh_call(z, st, gam, bet, ws, dinv):
    return pl.pallas_call(
        _xwh_kernel,
        grid=(NH, B),
        in_specs=[
            pl.BlockSpec((NV, DH), lambda h, g: (0, g)),
            pl.BlockSpec((2, DH), lambda h, g: (0, 0)),
            pl.BlockSpec((1, DH), lambda h, g: (0, 0)),
            pl.BlockSpec((1, DH), lambda h, g: (0, 0)),
            pl.BlockSpec((1, DH, DH), lambda h, g: (h, 0, 0)),
            pl.BlockSpec((NV, 1), lambda h, g: (0, 0)),
        ],
        out_specs=pl.BlockSpec((1, NV, DH), lambda h, g: (h, 0, g)),
        out_shape=jax.ShapeDtypeStruct((NH, NV, GF), jnp.bfloat16),
        interpret=_INTERPRET,
    )(z, st, gam, bet, ws, dinv)


def _agg_kernel(a_ref, y_ref, dinv_ref, bt_ref, z_ref, st_ref):
    jo = pl.program_id(0)
    part = jnp.dot(a_ref[...], y_ref[...], preferred_element_type=jnp.float32)
    z = part * dinv_ref[...] + bt_ref[...]
    z_ref[...] = z.astype(jnp.bfloat16)
    s = jnp.zeros((1, DH), jnp.float32)
    ss = jnp.zeros((1, DH), jnp.float32)
    for g in range(B):
        zz = z[:, DH * g:DH * (g + 1)]
        s = s + jnp.sum(zz, axis=0, keepdims=True)
        ss = ss + jnp.sum(zz * zz, axis=0, keepdims=True)
    st = jnp.concatenate([s, ss], axis=0)

    @pl.when(jo == 0)
    def _():
        st_ref[...] = st

    @pl.when(jo > 0)
    def _():
        st_ref[...] += st


def _agg_call(a1, y, dinv, bt):
    return pl.pallas_call(
        _agg_kernel,
        grid=(NV // AGG_RT,),
        in_specs=[
            pl.BlockSpec((AGG_RT, NV), lambda jo: (jo, 0)),
            pl.BlockSpec((NV, GF), lambda jo: (0, 0)),
            pl.BlockSpec((AGG_RT, 1), lambda jo: (jo, 0)),
            pl.BlockSpec((1, GF), lambda jo: (0, 0)),
        ],
        out_specs=[
            pl.BlockSpec((AGG_RT, GF), lambda jo: (jo, 0)),
            pl.BlockSpec((2, DH), lambda jo: (0, 0)),
        ],
        out_shape=[
            jax.ShapeDtypeStruct((NV, GF), jnp.bfloat16),
            jax.ShapeDtypeStruct((2, DH), jnp.float32),
        ],
        interpret=_INTERPRET,
    )(a1, y, dinv, bt)


def _aggh_kernel(a_ref, y_ref, dinv_ref, bt_ref, z_ref, st_ref, acc_ref):
    jo = pl.program_id(0)
    h = pl.program_id(1)
    part = jnp.dot(a_ref[...], y_ref[0], preferred_element_type=jnp.float32)
    z = part * dinv_ref[...] + bt_ref[0]
    z_ref[0] = z.astype(jnp.bfloat16)
    s = jnp.zeros((1, DH), jnp.float32)
    ss = jnp.zeros((1, DH), jnp.float32)
    for g in range(B):
        zz = z[:, DH * g:DH * (g + 1)]
        s = s + jnp.sum(zz, axis=0, keepdims=True)
        ss = ss + jnp.sum(zz * zz, axis=0, keepdims=True)
    st = jnp.concatenate([s, ss], axis=0)

    @pl.when(jo == 0)
    def _():
        acc_ref[h] = st

    @pl.when(jo > 0)
    def _():
        acc_ref[h] += st

    @pl.when(jo == NV // AGG_RT - 1)
    def _():
        st_ref[0] = acc_ref[h]


def _aggh_call(a1, yh, dinv, bts):
    return pl.pallas_call(
        _aggh_kernel,
        grid=(NV // AGG_RT, NH),
        in_specs=[
            pl.BlockSpec((AGG_RT, NV), lambda jo, h: (jo, 0)),
            pl.BlockSpec((1, NV, GF), lambda jo, h: (h, 0, 0)),
            pl.BlockSpec((AGG_RT, 1), lambda jo, h: (jo, 0)),
            pl.BlockSpec((1, 1, GF), lambda jo, h: (h, 0, 0)),
        ],
        out_specs=[
            pl.BlockSpec((1, AGG_RT, GF), lambda jo, h: (h, jo, 0)),
            pl.BlockSpec((1, 2, DH), lambda jo, h: (h, 0, 0)),
        ],
        out_shape=[
            jax.ShapeDtypeStruct((NH, NV, GF), jnp.bfloat16),
            jax.ShapeDtypeStruct((NH, 2, DH), jnp.float32),
        ],
        scratch_shapes=[pltpu.VMEM((NH, 2, DH), jnp.float32)],
        interpret=_INTERPRET,
    )(a1, yh, dinv, bts)


def _pool_kernel(z_ref, st_ref, gam_ref, bet_ref, fw_ref, fb_ref, o_ref):
    st = st_ref[0]
    mean = st[0:1] * (1.0 / NTOT)
    var = st[1:2] * (1.0 / NTOT) - mean * mean
    inv = lax.rsqrt(var + EPS)
    a = (z_ref[0].astype(jnp.float32) - mean) * (inv * gam_ref[0])
    a = jnp.maximum(a + bet_ref[0], 0.0)
    m = jnp.sum(a, axis=0, keepdims=True) * (1.0 / NV)
    logits = jnp.dot(m, fw_ref[0], preferred_element_type=jnp.float32)
    logits = logits + fb_ref[0]
    mx = jnp.max(logits, axis=1, keepdims=True)
    lse = mx + jnp.log(jnp.sum(jnp.exp(logits - mx), axis=1, keepdims=True))
    g = pl.program_id(1)
    o_ref[0, pl.ds(g, 1), :] = logits - lse


def _pool_call(zh, sth, gs, bes, fws, fbs):
    return pl.pallas_call(
        _pool_kernel,
        grid=(NH, B),
        in_specs=[
            pl.BlockSpec((1, NV, DH), lambda h, g: (h, 0, g)),
            pl.BlockSpec((1, 2, DH), lambda h, g: (h, 0, 0)),
            pl.BlockSpec((1, 1, DH), lambda h, g: (h, 0, 0)),
            pl.BlockSpec((1, 1, DH), lambda h, g: (h, 0, 0)),
            pl.BlockSpec((1, DH, NC), lambda h, g: (h, 0, 0)),
            pl.BlockSpec((1, 1, NC), lambda h, g: (h, 0, 0)),
        ],
        out_specs=pl.BlockSpec((1, B, NC), lambda h, g: (h, 0, 0)),
        out_shape=jax.ShapeDtypeStruct((NH, B, NC), jnp.float32),
        interpret=_INTERPRET,
    )(zh, sth, gs, bes, fws, fbs)


def kernel(x, edge_index, params):
    a1f = _build_adj(edge_index)
    dinv, a1 = _dinv_call(a1f)
    shared = params["shared"]

    p0 = shared[0]
    y = _xw_first(x.reshape(B, NV, 3), p0["W"], dinv)
    z, st = _agg_call(a1, y, dinv, jnp.tile(p0["b"], B)[None])
    gam, bet = p0["gamma"][None], p0["beta"][None]
    for p in shared[1:]:
        y = _xw_call(z, st, gam, bet, p["W"], dinv)
        z, st = _agg_call(a1, y, dinv, jnp.tile(p["b"], B)[None])
        gam, bet = p["gamma"][None], p["beta"][None]

    heads = [params["main"]] + list(params["auxs"])
    ws = jnp.stack([p["W"] for p in heads])
    bts = jnp.stack([jnp.tile(p["b"], B) for p in heads])[:, None, :]
    gs = jnp.stack([p["gamma"] for p in heads])[:, None, :]
    bes = jnp.stack([p["beta"] for p in heads])[:, None, :]
    fws = jnp.stack([p["fcW"] for p in heads])
    fbs = jnp.stack([p["fcb"] for p in heads])[:, None, :]

    yh = _xwh_call(z, st, gam, bet, ws, dinv)
    zh, sth = _aggh_call(a1, yh, dinv, bts)
    o_all = _pool_call(zh, sth, gs, bes, fws, fbs)
    return o_all[0], jnp.swapaxes(o_all[1:], 0, 1)


# speedup vs baseline: 1.3953x; 1.3953x over previous
import functools

import jax
import jax.numpy as jnp
from jax import lax
from jax.experimental import pallas as pl
from jax.experimental.pallas import tpu as pltpu
from jax.experimental.pallas import tpu_sc as plsc

B = 8
T = 144
V = 25
NV = T * V
NTOT = B * NV
DH = 128
NC = 60
EPS = 1e-5
E = 57600

GF = B * DH
NH = NUM_AUX_1 = 9
RB = 400
NB = NV // RB
AGG_RT = 1200
EDGE_TILE = 5760
SC_RB = 24
_UNROLL = 8
SC_NCH = NV // SC_RB
_INTERPRET = False


def _build_adj(edge_index):
    if _INTERPRET:
        src, dst = edge_index[0], edge_index[1]
        a = jnp.zeros((NV, NV), jnp.float32).at[dst, src].add(1.0)
        return a + jnp.eye(NV, dtype=jnp.float32)

    mesh = plsc.VectorSubcoreMesh(core_axis_name="c", subcore_axis_name="s")

    @functools.partial(
        pl.kernel,
        mesh=mesh,
        out_type=jax.ShapeDtypeStruct((NV, NV), jnp.float32),
        compiler_params=pltpu.CompilerParams(
            use_tc_tiling_on_sc=False, needs_layout_passes=False),
        scratch_types=[
            pltpu.VMEM((SC_RB, NV), jnp.float32),
            pltpu.VMEM((2, EDGE_TILE), jnp.int32),
            pltpu.VMEM((2, EDGE_TILE), jnp.int32),
            pltpu.SemaphoreType.DMA,
            pltpu.SemaphoreType.DMA,
        ],
    )
    def build(edge_hbm, zrow_hbm, out_hbm, ablk, srcb, dstb, sem0, sem1):
        wid = lax.axis_index("s") * 2 + lax.axis_index("c")
        ones16 = jnp.full((16,), 1.0, jnp.float32)
        lane = lax.iota(jnp.int32, 16)
        n_workers = 32
        sems = (sem0, sem1)
        n_tiles = E // EDGE_TILE
        for p in range((SC_NCH + 31) // 32):
            c = wid + n_workers * p

            @pl.when(c < SC_NCH)
            def _():
                lo = c * SC_RB
                pend = {}

                def start(t, b):
                    sl = pl.ds(t * EDGE_TILE, EDGE_TILE)
                    pend[b] = (
                        pltpu.async_copy(edge_hbm.at[0, sl], srcb.at[b],
                                         sems[b]),
                        pltpu.async_copy(edge_hbm.at[1, sl], dstb.at[b],
                                         sems[b]),
                    )

                start(0, 0)
                pltpu.sync_copy(zrow_hbm, ablk)
                for t in range(n_tiles):
                    b = t % 2
                    if t + 1 < n_tiles:
                        start(t + 1, 1 - b)
                    d1, d2 = pend[b]
                    d1.wait()
                    d2.wait()

                    def body(v, carry):
                        base = v * (16 * _UNROLL)
                        for u in range(_UNROLL):
                            dv = dstb[b, pl.ds(base + u * 16, 16)]
                            sv = srcb[b, pl.ds(base + u * 16, 16)]
                            m = (dv >= lo) & (dv < lo + SC_RB)
                            plsc.addupdate_scatter(
                                ablk, [dv - lo, sv], ones16, mask=m)
                        return carry

                    lax.fori_loop(0, EDGE_TILE // (16 * _UNROLL), body, 0)
                for q in range((SC_RB + 15) // 16):
                    rr = lane + q * 16
                    plsc.addupdate_scatter(
                        ablk, [rr, rr + lo], ones16, mask=rr < SC_RB)
                pltpu.sync_copy(ablk, out_hbm.at[pl.ds(lo, SC_RB)])

    zrow = jnp.zeros((SC_RB, NV), jnp.float32)
    return build(edge_index, zrow)


def _dinv_kernel(a_ref, o_ref, ab_ref):
    a = a_ref[...]
    deg = jnp.sum(a, axis=1, keepdims=True)
    o_ref[...] = lax.rsqrt(deg)
    ab_ref[...] = a.astype(jnp.bfloat16)


def _dinv_call(a1):
    return pl.pallas_call(
        _dinv_kernel,
        grid=(NB,),
        in_specs=[pl.BlockSpec((RB, NV), lambda j: (j, 0))],
        out_specs=[
            pl.BlockSpec((RB, 1), lambda j: (j, 0)),
            pl.BlockSpec((RB, NV), lambda j: (j, 0)),
        ],
        out_shape=[
            jax.ShapeDtypeStruct((NV, 1), jnp.float32),
            jax.ShapeDtypeStruct((NV, NV), jnp.bfloat16),
        ],
        interpret=_INTERPRET,
    )(a1)


def _xw_first_kernel(x_ref, w_ref, dinv_ref, o_ref):
    xb = x_ref[0]
    y = jnp.dot(xb, w_ref[...], preferred_element_type=jnp.float32)
    o_ref[...] = (y * dinv_ref[...]).astype(jnp.bfloat16)


def _xw_first(x, w, dinv):
    return pl.pallas_call(
        _xw_first_kernel,
        grid=(B,),
        in_specs=[
            pl.BlockSpec((1, NV, 3), lambda g: (g, 0, 0)),
            pl.BlockSpec((3, DH), lambda g: (0, 0)),
            pl.BlockSpec((NV, 1), lambda g: (0, 0)),
        ],
        out_specs=pl.BlockSpec((NV, DH), lambda g: (0, g)),
        out_shape=jax.ShapeDtypeStruct((NV, GF), jnp.bfloat16),
        interpret=_INTERPRET,
    )(x, w, dinv)


def _xw_kernel(z_ref, st_ref, gam_ref, bet_ref, w_ref, dinv_ref, o_ref):
    st = st_ref[...]
    mean = st[0:1] * (1.0 / NTOT)
    var = st[1:2] * (1.0 / NTOT) - mean * mean
    inv = lax.rsqrt(var + EPS)
    a = (z_ref[...].astype(jnp.float32) - mean) * (inv * gam_ref[...])
    a = jnp.maximum(a + bet_ref[...], 0.0)
    y = jnp.dot(a, w_ref[...], preferred_element_type=jnp.float32)
    o_ref[...] = (y * dinv_ref[...]).astype(jnp.bfloat16)


def _xw_call(z, st, gam, bet, w, dinv):
    return pl.pallas_call(
        _xw_kernel,
        grid=(B,),
        in_specs=[
            pl.BlockSpec((NV, DH), lambda g: (0, g)),
            pl.BlockSpec((2, DH), lambda g: (0, 0)),
            pl.BlockSpec((1, DH), lambda g: (0, 0)),
            pl.BlockSpec((1, DH), lambda g: (0, 0)),
            pl.BlockSpec((DH, DH), lambda g: (0, 0)),
            pl.BlockSpec((NV, 1), lambda g: (0, 0)),
        ],
        out_specs=pl.BlockSpec((NV, DH), lambda g: (0, g)),
        out_shape=jax.ShapeDtypeStruct((NV, GF), jnp.bfloat16),
        interpret=_INTERPRET,
    )(z, st, gam, bet, w, dinv)


def _xwh_kernel(z_ref, st_ref, gam_ref, bet_ref, w_ref, dinv_ref, o_ref):
    st = st_ref[...]
    mean = st[0:1] * (1.0 / NTOT)
    var = st[1:2] * (1.0 / NTOT) - mean * mean
    inv = lax.rsqrt(var + EPS)
    a = (z_ref[...].astype(jnp.float32) - mean) * (inv * gam_ref[...])
    a = jnp.maximum(a + bet_ref[...], 0.0)
    y = jnp.dot(a, w_ref[0], preferred_element_type=jnp.float32)
    y = pltpu.roll(y, lax.rem(NV - RB * pl.program_id(0), NV), axis=0)
    o_ref[0] = (y * dinv_ref[...]).astype(jnp.bfloat16)


def _xwh_call(z, st, gam, bet, ws, dinv):
    return pl.pallas_call(
        _xwh_kernel,
        grid=(NH, B),
        in_specs=[
            pl.BlockSpec((NV, DH), lambda h, g: (0, g)),
            pl.BlockSpec((2, DH), lambda h, g: (0, 0)),
            pl.BlockSpec((1, DH), lambda h, g: (0, 0)),
            pl.BlockSpec((1, DH), lambda h, g: (0, 0)),
            pl.BlockSpec((1, DH, DH), lambda h, g: (h, 0, 0)),
            pl.BlockSpec((NV, 1), lambda h, g: (0, 0)),
        ],
        out_specs=pl.BlockSpec((1, NV, DH), lambda h, g: (h, 0, g)),
        out_shape=jax.ShapeDtypeStruct((NH, NV, GF), jnp.bfloat16),
        interpret=_INTERPRET,
    )(z, st, gam, bet, ws, dinv)


def _agg_kernel(a_ref, y_ref, dinv_ref, bt_ref, z_ref, st_ref):
    jo = pl.program_id(0)
    part = jnp.dot(a_ref[...], y_ref[...], preferred_element_type=jnp.float32)
    z = part * dinv_ref[...] + bt_ref[...]
    z_ref[...] = z.astype(jnp.bfloat16)
    s = jnp.zeros((1, DH), jnp.float32)
    ss = jnp.zeros((1, DH), jnp.float32)
    for g in range(B):
        zz = z[:, DH * g:DH * (g + 1)]
        s = s + jnp.sum(zz, axis=0, keepdims=True)
        ss = ss + jnp.sum(zz * zz, axis=0, keepdims=True)
    st = jnp.concatenate([s, ss], axis=0)

    @pl.when(jo == 0)
    def _():
        st_ref[...] = st

    @pl.when(jo > 0)
    def _():
        st_ref[...] += st


def _agg_call(a1, y, dinv, bt):
    return pl.pallas_call(
        _agg_kernel,
        grid=(NV // AGG_RT,),
        in_specs=[
            pl.BlockSpec((AGG_RT, NV), lambda jo: (jo, 0)),
            pl.BlockSpec((NV, GF), lambda jo: (0, 0)),
            pl.BlockSpec((AGG_RT, 1), lambda jo: (jo, 0)),
            pl.BlockSpec((1, GF), lambda jo: (0, 0)),
        ],
        out_specs=[
            pl.BlockSpec((AGG_RT, GF), lambda jo: (jo, 0)),
            pl.BlockSpec((2, DH), lambda jo: (0, 0)),
        ],
        out_shape=[
            jax.ShapeDtypeStruct((NV, GF), jnp.bfloat16),
            jax.ShapeDtypeStruct((2, DH), jnp.float32),
        ],
        interpret=_INTERPRET,
    )(a1, y, dinv, bt)


def _aggh_kernel(a_ref, y_ref, dinv_ref, bt_ref, z_ref, st_ref, acc_ref):
    jo = pl.program_id(0)
    h = pl.program_id(1)
    part = jnp.dot(a_ref[...], y_ref[0], preferred_element_type=jnp.float32)
    z = part * dinv_ref[...] + bt_ref[0]
    z_ref[0] = z.astype(jnp.bfloat16)
    s = jnp.zeros((1, DH), jnp.float32)
    ss = jnp.zeros((1, DH), jnp.float32)
    for g in range(B):
        zz = z[:, DH * g:DH * (g + 1)]
        s = s + jnp.sum(zz, axis=0, keepdims=True)
        ss = ss + jnp.sum(zz * zz, axis=0, keepdims=True)
    st = jnp.concatenate([s, ss], axis=0)

    @pl.when(jo == 0)
    def _():
        acc_ref[h] = st

    @pl.when(jo > 0)
    def _():
        acc_ref[h] += st

    @pl.when(jo == NV // AGG_RT - 1)
    def _():
        st_ref[0] = acc_ref[h]


def _aggh_call(a1, yh, dinv, bts):
    return pl.pallas_call(
        _aggh_kernel,
        grid=(NV // AGG_RT, NH),
        in_specs=[
            pl.BlockSpec((AGG_RT, NV), lambda jo, h: (jo, 0)),
            pl.BlockSpec((1, NV, GF), lambda jo, h: (h, 0, 0)),
            pl.BlockSpec((AGG_RT, 1), lambda jo, h: (jo, 0)),
            pl.BlockSpec((1, 1, GF), lambda jo, h: (h, 0, 0)),
        ],
        out_specs=[
            pl.BlockSpec((1, AGG_RT, GF), lambda jo, h: (h, jo, 0)),
            pl.BlockSpec((1, 2, DH), lambda jo, h: (h, 0, 0)),
        ],
        out_shape=[
            jax.ShapeDtypeStruct((NH, NV, GF), jnp.bfloat16),
            jax.ShapeDtypeStruct((NH, 2, DH), jnp.float32),
        ],
        scratch_shapes=[pltpu.VMEM((NH, 2, DH), jnp.float32)],
        interpret=_INTERPRET,
    )(a1, yh, dinv, bts)


def _pool_kernel(z_ref, st_ref, gam_ref, bet_ref, fw_ref, fb_ref, o_ref):
    st = st_ref[0]
    mean = st[0:1] * (1.0 / NTOT)
    var = st[1:2] * (1.0 / NTOT) - mean * mean
    inv = lax.rsqrt(var + EPS)
    a = (z_ref[0].astype(jnp.float32) - mean) * (inv * gam_ref[0])
    a = jnp.maximum(a + bet_ref[0], 0.0)
    m = jnp.sum(a, axis=0, keepdims=True) * (1.0 / NV)
    logits = jnp.dot(m, fw_ref[0], preferred_element_type=jnp.float32)
    logits = logits + fb_ref[0]
    mx = jnp.max(logits, axis=1, keepdims=True)
    lse = mx + jnp.log(jnp.sum(jnp.exp(logits - mx), axis=1, keepdims=True))
    g = pl.program_id(1)
    o_ref[0, pl.ds(g, 1), :] = logits - lse


def _pool_call(zh, sth, gs, bes, fws, fbs):
    return pl.pallas_call(
        _pool_kernel,
        grid=(NH, B),
        in_specs=[
            pl.BlockSpec((1, NV, DH), lambda h, g: (h, 0, g)),
            pl.BlockSpec((1, 2, DH), lambda h, g: (h, 0, 0)),
            pl.BlockSpec((1, 1, DH), lambda h, g: (h, 0, 0)),
            pl.BlockSpec((1, 1, DH), lambda h, g: (h, 0, 0)),
            pl.BlockSpec((1, DH, NC), lambda h, g: (h, 0, 0)),
            pl.BlockSpec((1, 1, NC), lambda h, g: (h, 0, 0)),
        ],
        out_specs=pl.BlockSpec((1, B, NC), lambda h, g: (h, 0, 0)),
        out_shape=jax.ShapeDtypeStruct((NH, B, NC), jnp.float32),
        interpret=_INTERPRET,
    )(zh, sth, gs, bes, fws, fbs)


def kernel(x, edge_index, params):
    a1f = _build_adj(edge_index)
    dinv, a1 = _dinv_call(a1f)
    shared = params["shared"]

    p0 = shared[0]
    y = _xw_first(x.reshape(B, NV, 3), p0["W"], dinv)
    z, st = _agg_call(a1, y, dinv, jnp.tile(p0["b"], B)[None])
    gam, bet = p0["gamma"][None], p0["beta"][None]
    for p in shared[1:]:
        y = _xw_call(z, st, gam, bet, p["W"], dinv)
        z, st = _agg_call(a1, y, dinv, jnp.tile(p["b"], B)[None])
        gam, bet = p["gamma"][None], p["beta"][None]

    heads = [params["main"]] + list(params["auxs"])
    ws = jnp.stack([p["W"] for p in heads])
    bts = jnp.stack([jnp.tile(p["b"], B) for p in heads])[:, None, :]
    gs = jnp.stack([p["gamma"] for p in heads])[:, None, :]
    bes = jnp.stack([p["beta"] for p in heads])[:, None, :]
    fws = jnp.stack([p["fcW"] for p in heads])
    fbs = jnp.stack([p["fcb"] for p in heads])[:, None, :]

    yh = _xwh_call(z, st, gam, bet, ws, dinv)
    zh, sth = _aggh_call(a1, yh, dinv, bts)
    o_all = _pool_call(zh, sth, gs, bes, fws, fbs)
    return o_all[0], jnp.swapaxes(o_all[1:], 0, 1)
